# baseline (speedup 1.0000x reference)
import functools

import jax
import jax.numpy as jnp
from jax import lax
from jax.experimental import pallas as pl
from jax.experimental.pallas import tpu as pltpu
from jax.experimental.pallas import tpu_sc as plsc

DIM = 64
NEG = 10
NC = 2
NS = 16
NW = NC * NS
LANES = 16
SROW = 2 * DIM


def _sc_v_gather(v_weight, v_pos, v_neg_flat, batch):
    chunk = 128
    bpw = batch // NW
    nchunks = bpw // chunk
    mesh = plsc.VectorSubcoreMesh(
        core_axis_name="c", subcore_axis_name="s", num_cores=NC, num_subcores=NS
    )

    @functools.partial(
        pl.kernel,
        out_type=[
            jax.ShapeDtypeStruct((batch, DIM), jnp.float32),
            jax.ShapeDtypeStruct((batch, DIM), jnp.float32),
        ],
        mesh=mesh,
        compiler_params=pltpu.CompilerParams(
            needs_layout_passes=False, use_tc_tiling_on_sc=False),
        scratch_types=[
            pltpu.VMEM((chunk,), jnp.int32),
            pltpu.VMEM((NEG * chunk,), jnp.int32),
            pltpu.VMEM((chunk, DIM), jnp.float32),
            pltpu.VMEM((NEG * chunk, DIM), jnp.float32),
            pltpu.VMEM((chunk, DIM), jnp.float32),
            pltpu.SemaphoreType.DMA,
        ],
    )
    def v_kernel(v_w, vp, vn, vpos_out, vsum_out,
                 idx_v, idx_n, rows_v, rows_n, nsum, sem):
        wid = lax.axis_index("s") * NC + lax.axis_index("c")
        base = wid * bpw
        for c in range(nchunks):
            off = base + c * chunk
            pltpu.sync_copy(vp.at[pl.ds(off, chunk)], idx_v)
            pltpu.sync_copy(vn.at[pl.ds(off * NEG, chunk * NEG)], idx_n)
            cps = [pltpu.async_copy(v_w.at[idx_v], rows_v, sem)]
            for j in range(NEG):
                cps.append(
                    pltpu.async_copy(
                        v_w.at[idx_n.at[pl.ds(j * chunk, chunk)]],
                        rows_n.at[pl.ds(j * chunk, chunk)],
                        sem,
                    )
                )
            for cp in cps:
                cp.wait()

            def body(b, _):
                for k in range(DIM // LANES):
                    sl = pl.ds(k * LANES, LANES)
                    acc = rows_n[b * NEG, sl]
                    for n in range(1, NEG):
                        acc = acc + rows_n[b * NEG + n, sl]
                    nsum[b, sl] = acc
                return 0

            lax.fori_loop(0, chunk, body, 0)
            pltpu.sync_copy(rows_v, vpos_out.at[pl.ds(off, chunk)])
            pltpu.sync_copy(nsum, vsum_out.at[pl.ds(off, chunk)])

    return v_kernel(v_weight, v_pos, v_neg_flat)


def _sc_u_dots(u_w2, u_pos, vpos_rows, vsum_rows, batch):
    half = u_w2.shape[0]
    chunk = 64
    bpw = batch // NW
    nchunks = bpw // chunk
    mesh = plsc.VectorSubcoreMesh(
        core_axis_name="c", subcore_axis_name="s", num_cores=NC, num_subcores=NS
    )

    @functools.partial(
        pl.kernel,
        out_type=[
            jax.ShapeDtypeStruct((batch,), jnp.float32),
            jax.ShapeDtypeStruct((batch,), jnp.float32),
        ],
        mesh=mesh,
        compiler_params=pltpu.CompilerParams(needs_layout_passes=False),
        scratch_types=[
            pltpu.VMEM((chunk,), jnp.int32),
            pltpu.VMEM((chunk,), jnp.int32),
            pltpu.VMEM((chunk, SROW), jnp.float32),
            pltpu.VMEM((chunk, DIM), jnp.float32),
            pltpu.VMEM((chunk, DIM), jnp.float32),
            pltpu.VMEM((chunk,), jnp.float32),
            pltpu.VMEM((chunk,), jnp.float32),
            pltpu.SemaphoreType.DMA,
        ],
    )
    def u_kernel(u_w, up, vpr, vsr, score_out, nscore_out,
                 idx_u, div_u, rows_u, rows_vp, rows_vs,
                 sc_chunk, nc_chunk, sem):
        wid = lax.axis_index("s") * NC + lax.axis_index("c")
        base = wid * bpw
        lane_iota = lax.iota(jnp.int32, LANES)
        for c in range(nchunks):
            off = base + c * chunk
            pltpu.sync_copy(up.at[pl.ds(off, chunk)], idx_u)
            for t in range(chunk // LANES):
                sl = pl.ds(t * LANES, LANES)
                x = idx_u[sl]
                div_u[sl] = x - jnp.where(x >= half, half, 0)
            cps = [
                pltpu.async_copy(u_w.at[div_u], rows_u, sem),
                pltpu.async_copy(vpr.at[pl.ds(off, chunk)], rows_vp, sem),
                pltpu.async_copy(vsr.at[pl.ds(off, chunk)], rows_vs, sem),
            ]
            for cp in cps:
                cp.wait()

            for g in range(chunk // LANES):
                gsl = pl.ds(g * LANES, LANES)
                rowp = g * LANES + lane_iota
                cu = jnp.where(idx_u[gsl] >= half, DIM, 0)

                def dloop(d, carry):
                    acc_s, acc_n = carry
                    dv = jnp.full((LANES,), 0, jnp.int32) + d
                    gu = plsc.load_gather(rows_u, [rowp, cu + dv])
                    gvp = plsc.load_gather(rows_vp, [rowp, dv])
                    gvs = plsc.load_gather(rows_vs, [rowp, dv])
                    return (acc_s + gu * gvp, acc_n + gu * gvs)

                zeros = jnp.zeros((LANES,), jnp.float32)
                acc_s, acc_n = lax.fori_loop(0, DIM, dloop, (zeros, zeros))
                sc_chunk[gsl] = acc_s
                nc_chunk[gsl] = acc_n

            pltpu.sync_copy(sc_chunk, score_out.at[pl.ds(off, chunk)])
            pltpu.sync_copy(nc_chunk, nscore_out.at[pl.ds(off, chunk)])

    return u_kernel(u_w2, u_pos, vpos_rows, vsum_rows)


def _compact_body(lo_ref, hi_ref, out_ref):
    d = lo_ref.shape[1]
    out_ref[:, 0:d] = lo_ref[...]
    out_ref[:, d:2 * d] = hi_ref[...]


def _compact(table):
    vocab, d = table.shape
    br = 4096
    nblk = vocab // 2 // br
    return pl.pallas_call(
        _compact_body,
        grid=(nblk,),
        in_specs=[
            pl.BlockSpec((br, d), lambda i: (i, 0)),
            pl.BlockSpec((br, d), lambda i, n=nblk: (i + n, 0)),
        ],
        out_specs=pl.BlockSpec((br, 2 * d), lambda i: (i, 0)),
        out_shape=jax.ShapeDtypeStruct((vocab // 2, 2 * d), jnp.float32),
    )(table, table)


def _tc_loss_body(s_ref, n_ref, o_ref):
    s = s_ref[...]
    n = n_ref[...]
    val = jax.nn.log_sigmoid(s) + jax.nn.log_sigmoid(-n)
    o_ref[0, 0] = -jnp.sum(val)


def kernel(u_pos, v_pos, v_neg, batch_size, u_weight, v_weight):
    batch = u_pos.shape[0]
    half = u_weight.shape[0] // 2
    u_w2 = jnp.concatenate([u_weight[:half], u_weight[half:]], axis=1)
    vpos_rows, vsum_rows = _sc_v_gather(
        v_weight,
        v_pos.astype(jnp.int32),
        v_neg.reshape(-1).astype(jnp.int32),
        batch,
    )
    scores, nscores = _sc_u_dots(
        u_w2, u_pos.astype(jnp.int32), vpos_rows, vsum_rows, batch)
    rows = batch // 128
    loss_sum = pl.pallas_call(
        _tc_loss_body,
        out_shape=jax.ShapeDtypeStruct((1, 1), jnp.float32),
        out_specs=pl.BlockSpec(memory_space=pltpu.SMEM),
    )(scores.reshape(rows, 128), nscores.reshape(rows, 128))
    return loss_sum[0, 0] / batch_size


# speedup vs baseline: 1.4537x; 1.4537x over previous
import functools

import jax
import jax.numpy as jnp
from jax import lax
from jax.experimental import pallas as pl
from jax.experimental.pallas import tpu as pltpu
from jax.experimental.pallas import tpu_sc as plsc

DIM = 64
NEG = 10
NC = 2
NS = 16
NW = NC * NS
LANES = 16
GRP = 8
CHUNK = 16


def _sc_scores(u_w3, v_w3, u_pos, v_pos, v_neg_flat, batch):
    bpw = batch // NW
    nchunks = bpw // CHUNK
    mesh = plsc.VectorSubcoreMesh(
        core_axis_name="c", subcore_axis_name="s", num_cores=NC, num_subcores=NS
    )

    @functools.partial(
        pl.kernel,
        out_type=[
            jax.ShapeDtypeStruct((batch,), jnp.float32),
            jax.ShapeDtypeStruct((batch,), jnp.float32),
        ],
        mesh=mesh,
        compiler_params=pltpu.CompilerParams(needs_layout_passes=False),
        scratch_types=[
            pltpu.VMEM((CHUNK,), jnp.int32),
            pltpu.VMEM((CHUNK,), jnp.int32),
            pltpu.VMEM((NEG * CHUNK,), jnp.int32),
            pltpu.VMEM((CHUNK, GRP, DIM), jnp.float32),
            pltpu.VMEM((CHUNK, GRP, DIM), jnp.float32),
            pltpu.VMEM((NEG * CHUNK // 2, GRP, DIM), jnp.float32),
            pltpu.VMEM((CHUNK,), jnp.float32),
            pltpu.VMEM((CHUNK,), jnp.float32),
            pltpu.SemaphoreType.DMA,
        ],
    )
    def sc_kernel(u_w, v_w, up, vp, vn, score_out, nscore_out,
                  idx_u, idx_v, idx_n,
                  rows_u, rows_v, rows_n, sc_chunk, nc_chunk, sem):
        wid = lax.axis_index("s") * NC + lax.axis_index("c")
        base = wid * bpw
        lane_iota = lax.iota(jnp.int32, LANES)

        def chunk_body(c, _):
            off = base + c * CHUNK
            pltpu.sync_copy(up.at[pl.ds(off, CHUNK)], idx_u)
            pltpu.sync_copy(vp.at[pl.ds(off, CHUNK)], idx_v)
            pltpu.sync_copy(vn.at[pl.ds(off * NEG, CHUNK * NEG)], idx_n)

            qu = idx_u[...] >> 3
            qv = idx_v[...] >> 3
            for j in range(CHUNK):
                pltpu.async_copy(u_w.at[qu[j]], rows_u.at[j], sem)
                pltpu.async_copy(v_w.at[qv[j]], rows_v.at[j], sem)

            nh = NEG // 2

            def issue_negs(h):
                for t in range(nh):
                    qn = plsc.load_gather(
                        idx_n, [lane_iota * NEG + (h * nh + t)]) >> 3
                    for j in range(LANES):
                        pltpu.async_copy(
                            v_w.at[qn[j]], rows_n.at[t * LANES + j], sem)

            def drain(k):
                for _ in range(k):
                    pltpu.make_async_copy(
                        u_w.at[pl.ds(0, CHUNK)],
                        rows_n.at[pl.ds(0, CHUNK)], sem).wait()

            ru = idx_u[...] & 7
            rv = idx_v[...] & 7

            def make_neg_meta(h):
                rows = []
                for n in range(nh):
                    ni = plsc.load_gather(
                        idx_n, [lane_iota * NEG + (h * nh + n)])
                    rows.append(ni & 7)
                return rows

            def dpass(h, acc_s, acc_n, with_uv):
                nrow = make_neg_meta(h)

                def dloop(d, carry):
                    a_s, a_n = carry
                    dv = jnp.full((LANES,), 0, jnp.int32) + d
                    gu = plsc.load_gather(rows_u, [lane_iota, ru, dv])
                    gn = None
                    for n in range(nh):
                        gx = plsc.load_gather(
                            rows_n, [n * LANES + lane_iota, nrow[n], dv])
                        gn = gx if gn is None else gn + gx
                    if with_uv:
                        gv = plsc.load_gather(rows_v, [lane_iota, rv, dv])
                        a_s = a_s + gu * gv
                    return (a_s, a_n + gu * gn)

                return lax.fori_loop(0, DIM, dloop, (acc_s, acc_n))

            zeros = jnp.zeros((LANES,), jnp.float32)
            issue_negs(0)
            drain(nh + 2)
            acc_s, acc_n = dpass(0, zeros, zeros, with_uv=True)
            issue_negs(1)
            drain(nh)
            acc_s, acc_n = dpass(1, acc_s, acc_n, with_uv=False)
            sc_chunk[...] = acc_s
            nc_chunk[...] = acc_n

            pltpu.sync_copy(sc_chunk, score_out.at[pl.ds(off, CHUNK)])
            pltpu.sync_copy(nc_chunk, nscore_out.at[pl.ds(off, CHUNK)])
            return 0

        lax.fori_loop(0, nchunks, chunk_body, 0)

    return sc_kernel(u_w3, v_w3, u_pos, v_pos, v_neg_flat)


def _tc_loss_body(s_ref, n_ref, o_ref):
    s = s_ref[...]
    n = n_ref[...]
    val = jax.nn.log_sigmoid(s) + jax.nn.log_sigmoid(-n)
    o_ref[0, 0] = -jnp.sum(val)


def kernel(u_pos, v_pos, v_neg, batch_size, u_weight, v_weight):
    batch = u_pos.shape[0]
    vocab = u_weight.shape[0]
    u_w3 = u_weight.reshape(vocab // GRP, GRP, DIM)
    v_w3 = v_weight.reshape(vocab // GRP, GRP, DIM)
    scores, nscores = _sc_scores(
        u_w3,
        v_w3,
        u_pos.astype(jnp.int32),
        v_pos.astype(jnp.int32),
        v_neg.reshape(-1).astype(jnp.int32),
        batch,
    )
    rows = batch // 128
    loss_sum = pl.pallas_call(
        _tc_loss_body,
        out_shape=jax.ShapeDtypeStruct((1, 1), jnp.float32),
        out_specs=pl.BlockSpec(memory_space=pltpu.SMEM),
    )(scores.reshape(rows, 128), nscores.reshape(rows, 128))
    return loss_sum[0, 0] / batch_size
